# Initial kernel scaffold; baseline (speedup 1.0000x reference)
import jax
import jax.numpy as jnp
from jax.experimental import pallas as pl


def kernel(input_vect, weights, epoch):
    raise NotImplementedError("write your pallas kernel here")


# speedup vs baseline: 17.0777x; 17.0777x over previous
import jax
import jax.numpy as jnp
from jax import lax
from jax.experimental import pallas as pl
from jax.experimental.pallas import tpu as pltpu

_M = 32
_N = 32
_MAX_EPOCHS = 100
_INITIAL_RADIUS = max(_M, _N) / 2.0
_INITIAL_LR = 0.1
_STD_COEFF = 0.5


def _som_body(params_ref, x_ref, w_ref, loc_ref, out_w_ref):
    x = x_ref[...]
    w = w_ref[...]
    alpha = params_ref[0]
    neg_inv_two_sigma2 = params_ref[1]

    B = x.shape[0]
    K = w.shape[0]

    wT = w.T
    wn = jnp.sum(wT * wT, axis=0)
    xw = lax.dot_general(
        x, wT, (((1,), (0,)), ((), ())),
        precision=lax.Precision.HIGHEST,
        preferred_element_type=jnp.float32,
    )
    d = wn[None, :] - 2.0 * xw

    kk = lax.broadcasted_iota(jnp.int32, (B, K), 1)
    i1 = jnp.argmin(d, axis=1).astype(jnp.int32)
    d_masked = jnp.where(kk == i1[:, None], jnp.float32(1e30), d)
    i2 = jnp.argmin(d_masked, axis=1).astype(jnp.int32)

    oh1 = (kk == i1[:, None]).astype(jnp.float32)
    oh2 = (kk == i2[:, None]).astype(jnp.float32)
    g1 = lax.dot_general(
        oh1, w, (((1,), (0,)), ((), ())),
        precision=lax.Precision.HIGHEST,
        preferred_element_type=jnp.float32,
    )
    g2 = lax.dot_general(
        oh2, w, (((1,), (0,)), ((), ())),
        precision=lax.Precision.HIGHEST,
        preferred_element_type=jnp.float32,
    )
    e1 = jnp.sum((g1 - x) ** 2, axis=1)
    e2 = jnp.sum((g2 - x) ** 2, axis=1)
    take2 = (e2 < e1) | ((e2 == e1) & (i2 < i1))
    bmu = jnp.where(take2, i2, i1)

    bi = bmu // _N
    bj = bmu - bi * _N
    u = lax.broadcasted_iota(jnp.int32, (K, B), 0) // _N
    v = lax.broadcasted_iota(jnp.int32, (K, B), 0) - u * _N
    grid_d2 = (u - bi[None, :]) ** 2 + (v - bj[None, :]) ** 2
    lrT = alpha * jnp.exp(grid_d2.astype(jnp.float32) * neg_inv_two_sigma2)

    num = lax.dot_general(
        lrT, x, (((1,), (0,)), ((), ())),
        precision=lax.Precision.HIGHEST,
        preferred_element_type=jnp.float32,
    )
    den = jnp.sum(lrT, axis=1) + 1e-12
    out_w_ref[...] = num / den[:, None]
    loc_ref[...] = jnp.concatenate([bi[None, :], bj[None, :]], axis=0)


def kernel(input_vect, weights, epoch):
    B, D = input_vect.shape
    K = weights.shape[0]

    epoch_f = jnp.asarray(epoch, jnp.float32)
    radius = _INITIAL_RADIUS - epoch_f * (
        (_INITIAL_RADIUS - 1.0) / float(_MAX_EPOCHS - 1)
    )
    alpha = _INITIAL_LR * (1.0 - epoch_f / float(_MAX_EPOCHS))
    neg_inv_two_sigma2 = -1.0 / (2.0 * jnp.square(radius * _STD_COEFF))
    params = jnp.stack([alpha, neg_inv_two_sigma2]).astype(jnp.float32)

    loc2, new_weights = pl.pallas_call(
        _som_body,
        out_shape=(
            jax.ShapeDtypeStruct((2, B), jnp.int32),
            jax.ShapeDtypeStruct((K, D), jnp.float32),
        ),
        in_specs=[
            pl.BlockSpec(memory_space=pltpu.SMEM),
            pl.BlockSpec(memory_space=pltpu.VMEM),
            pl.BlockSpec(memory_space=pltpu.VMEM),
        ],
        out_specs=(
            pl.BlockSpec(memory_space=pltpu.VMEM),
            pl.BlockSpec(memory_space=pltpu.VMEM),
        ),
    )(params, input_vect, weights)

    bmu_locs = loc2.T
    return bmu_locs, new_weights


# speedup vs baseline: 25.2820x; 1.4804x over previous
import jax
import jax.numpy as jnp
from jax import lax
from jax.experimental import pallas as pl
from jax.experimental.pallas import tpu as pltpu

_M = 32
_N = 32
_MAX_EPOCHS = 100
_INITIAL_RADIUS = max(_M, _N) / 2.0
_INITIAL_LR = 0.1
_STD_COEFF = 0.5


def _som_body(params_ref, x_ref, w_ref, loc_ref, out_w_ref):
    x = x_ref[...]
    w = w_ref[...]
    alpha = params_ref[0]
    neg_inv_two_sigma2 = params_ref[1]

    B = x.shape[0]
    K = w.shape[0]

    wT = w.T
    wn = jnp.sum(wT * wT, axis=0)

    xh = x.astype(jnp.bfloat16).astype(jnp.float32)
    xl = x - xh
    wTh = wT.astype(jnp.bfloat16).astype(jnp.float32)
    wTl = wT - wTh

    def _dot(a, b):
        return lax.dot_general(
            a, b, (((1,), (0,)), ((), ())),
            preferred_element_type=jnp.float32,
        )

    xw = _dot(xh, wTh) + (_dot(xh, wTl) + _dot(xl, wTh))
    d = wn[None, :] - 2.0 * xw

    kk = lax.broadcasted_iota(jnp.int32, (B, K), 1)
    i1 = jnp.argmin(d, axis=1).astype(jnp.int32)
    d_masked = jnp.where(kk == i1[:, None], jnp.float32(1e30), d)
    i2 = jnp.argmin(d_masked, axis=1).astype(jnp.int32)

    oh1 = (kk == i1[:, None]).astype(jnp.float32)
    oh2 = (kk == i2[:, None]).astype(jnp.float32)
    w1 = w.astype(jnp.bfloat16).astype(jnp.float32)
    r1 = w - w1
    w2 = r1.astype(jnp.bfloat16).astype(jnp.float32)
    w3 = r1 - w2
    g1 = _dot(oh1, w1) + (_dot(oh1, w2) + _dot(oh1, w3))
    g2 = _dot(oh2, w1) + (_dot(oh2, w2) + _dot(oh2, w3))
    e1 = jnp.sum((g1 - x) ** 2, axis=1)
    e2 = jnp.sum((g2 - x) ** 2, axis=1)
    take2 = (e2 < e1) | ((e2 == e1) & (i2 < i1))
    bmu = jnp.where(take2, i2, i1)

    bi = bmu // _N
    bj = bmu - bi * _N
    ui = lax.broadcasted_iota(jnp.int32, (_M, B), 0)
    fa = alpha * jnp.exp(
        ((ui - bi[None, :]) ** 2).astype(jnp.float32) * neg_inv_two_sigma2
    )
    fb = jnp.exp(
        ((ui - bj[None, :]) ** 2).astype(jnp.float32) * neg_inv_two_sigma2
    )
    lrT = (fa[:, None, :] * fb[None, :, :]).reshape(K, B)

    num = lax.dot_general(
        lrT, x, (((1,), (0,)), ((), ())),
        preferred_element_type=jnp.float32,
    )
    den = jnp.sum(lrT, axis=1) + 1e-12
    out_w_ref[...] = num / den[:, None]
    loc_ref[...] = jnp.concatenate([bi[None, :], bj[None, :]], axis=0)


def kernel(input_vect, weights, epoch):
    B, D = input_vect.shape
    K = weights.shape[0]

    epoch_f = jnp.asarray(epoch, jnp.float32)
    radius = _INITIAL_RADIUS - epoch_f * (
        (_INITIAL_RADIUS - 1.0) / float(_MAX_EPOCHS - 1)
    )
    alpha = _INITIAL_LR * (1.0 - epoch_f / float(_MAX_EPOCHS))
    neg_inv_two_sigma2 = -1.0 / (2.0 * jnp.square(radius * _STD_COEFF))
    params = jnp.stack([alpha, neg_inv_two_sigma2]).astype(jnp.float32)

    loc2, new_weights = pl.pallas_call(
        _som_body,
        out_shape=(
            jax.ShapeDtypeStruct((2, B), jnp.int32),
            jax.ShapeDtypeStruct((K, D), jnp.float32),
        ),
        in_specs=[
            pl.BlockSpec(memory_space=pltpu.SMEM),
            pl.BlockSpec(memory_space=pltpu.VMEM),
            pl.BlockSpec(memory_space=pltpu.VMEM),
        ],
        out_specs=(
            pl.BlockSpec(memory_space=pltpu.VMEM),
            pl.BlockSpec(memory_space=pltpu.VMEM),
        ),
    )(params, input_vect, weights)

    bmu_locs = loc2.T
    return bmu_locs, new_weights
